# baseline (speedup 1.0000x reference)
import jax
import jax.numpy as jnp
from jax import lax
from jax.experimental import pallas as pl
from jax.experimental.pallas import tpu as pltpu
from jax.experimental.pallas import tpu_sc as plsc

N_NODES = 10000
D = 256
N_EDGES = 160000
NC = 2
NS = 16
NW = NC * NS
EPW = 5120
PAD_E = EPW * NW
C = 64
NCHUNK = EPW // C
LG = 16
NSEG = D // LG
NGRP = C // LG


def _permute(x, idx):
    dnums = lax.GatherDimensionNumbers(
        offset_dims=(), collapsed_slice_dims=(0,), start_index_map=(0,))
    return lax.gather(x, idx[:, None], dnums, (1,),
                      mode=lax.GatherScatterMode.PROMISE_IN_BOUNDS)


def _body(z_hbm, src_hbm, dst_hbm, out_hbm,
          sidx, didx, srow, drow, outv,
          sem_s0, sem_s1, sem_d0, sem_d1):
    wid = lax.axis_index("s") * NC + lax.axis_index("c")
    base = wid * EPW

    pltpu.sync_copy(src_hbm.at[pl.ds(base, EPW)], sidx)
    pltpu.sync_copy(dst_hbm.at[pl.ds(base, EPW)], didx)

    lane = lax.iota(jnp.int32, LG)
    perms = [(lane + s) & (LG - 1) for s in (8, 4, 2, 1)]
    sems = ((sem_s0, sem_d0), (sem_s1, sem_d1))

    def start(g, b):
        pltpu.async_copy(z_hbm.at[sidx.at[pl.ds(g * C, C)]], srow.at[b],
                         sems[b][0])
        pltpu.async_copy(z_hbm.at[didx.at[pl.ds(g * C, C)]], drow.at[b],
                         sems[b][1])

    def wait(g, b):
        pltpu.make_async_copy(z_hbm.at[sidx.at[pl.ds(g * C, C)]], srow.at[b],
                              sems[b][0]).wait()
        pltpu.make_async_copy(z_hbm.at[didx.at[pl.ds(g * C, C)]], drow.at[b],
                              sems[b][1]).wait()

    def compute(g, b):
        for q in range(NGRP):
            gvec = jnp.zeros((LG,), jnp.float32)
            for i in range(LG):
                e = q * LG + i
                accs = [srow[b, e, pl.ds(a * LG, LG)] * drow[b, e, pl.ds(a * LG, LG)]
                        for a in range(4)]
                for j in range(4, NSEG):
                    a = j & 3
                    accs[a] = accs[a] + (srow[b, e, pl.ds(j * LG, LG)]
                                         * drow[b, e, pl.ds(j * LG, LG)])
                acc = (accs[0] + accs[1]) + (accs[2] + accs[3])
                for p in perms:
                    acc = acc + _permute(acc, p)
                gvec = jnp.where(lane == i, acc, gvec)
            outv[pl.ds(g * C + q * LG, LG)] = 1.0 / (1.0 + jnp.exp(-gvec))

    start(0, 0)
    start(1, 1)

    def outer(t, carry):
        for b in range(2):
            g = 2 * t + b
            wait(g, b)
            compute(g, b)

            @pl.when(g + 2 < NCHUNK)
            def _():
                start(g + 2, b)
        return carry

    lax.fori_loop(0, NCHUNK // 2, outer, 0)
    pltpu.sync_copy(outv, out_hbm.at[pl.ds(base, EPW)])


@jax.jit
def _decode(z, src, dst):
    mesh = plsc.VectorSubcoreMesh(core_axis_name="c", subcore_axis_name="s")
    f = pl.kernel(
        _body,
        mesh=mesh,
        out_type=jax.ShapeDtypeStruct((PAD_E,), jnp.float32),
        scratch_types=[
            pltpu.VMEM((EPW,), jnp.int32),
            pltpu.VMEM((EPW,), jnp.int32),
            pltpu.VMEM((2, C, D), jnp.float32),
            pltpu.VMEM((2, C, D), jnp.float32),
            pltpu.VMEM((EPW,), jnp.float32),
            pltpu.SemaphoreType.DMA,
            pltpu.SemaphoreType.DMA,
            pltpu.SemaphoreType.DMA,
            pltpu.SemaphoreType.DMA,
        ],
    )
    return f(z, src, dst)


def kernel(z, edge_index):
    src = jnp.pad(edge_index[0].astype(jnp.int32), (0, PAD_E - N_EDGES))
    dst = jnp.pad(edge_index[1].astype(jnp.int32), (0, PAD_E - N_EDGES))
    return _decode(z, src, dst)[:N_EDGES]


# speedup vs baseline: 1.2845x; 1.2845x over previous
import jax
import jax.numpy as jnp
from jax import lax
from jax.experimental import pallas as pl
from jax.experimental.pallas import tpu as pltpu
from jax.experimental.pallas import tpu_sc as plsc

N_NODES = 10000
D = 256
N_EDGES = 160000
NC = 2
NS = 16
NW = NC * NS
EPW = 5120
PAD_E = EPW * NW
C = 16
NCHUNK = EPW // C
LG = 16
NSEG = D // LG


def _permute(x, idx):
    dnums = lax.GatherDimensionNumbers(
        offset_dims=(), collapsed_slice_dims=(0,), start_index_map=(0,))
    return lax.gather(x, idx[:, None], dnums, (1,),
                      mode=lax.GatherScatterMode.PROMISE_IN_BOUNDS)


def _body(z_hbm, src_hbm, dst_hbm, out_hbm,
          sidx, didx, srow, drow, outv,
          sem_s0, sem_s1, sem_d0, sem_d1):
    wid = lax.axis_index("s") * NC + lax.axis_index("c")
    base = wid * EPW

    pltpu.sync_copy(src_hbm.at[pl.ds(base, EPW)], sidx)
    pltpu.sync_copy(dst_hbm.at[pl.ds(base, EPW)], didx)

    lane = lax.iota(jnp.int32, LG)
    perms = [(lane + s) & (LG - 1) for s in (8, 4, 2, 1)]
    sems = ((sem_s0, sem_d0), (sem_s1, sem_d1))

    def start(g, b):
        pltpu.async_copy(z_hbm.at[sidx.at[pl.ds(g * C, C)]], srow.at[b],
                         sems[b][0])
        pltpu.async_copy(z_hbm.at[didx.at[pl.ds(g * C, C)]], drow.at[b],
                         sems[b][1])

    def wait(g, b):
        pltpu.make_async_copy(z_hbm.at[sidx.at[pl.ds(g * C, C)]], srow.at[b],
                              sems[b][0]).wait()
        pltpu.make_async_copy(z_hbm.at[didx.at[pl.ds(g * C, C)]], drow.at[b],
                              sems[b][1]).wait()

    def compute(g, b):
        gvec = jnp.zeros((LG,), jnp.float32)
        for e in range(C):
            accs = [srow[b, e, pl.ds(a * LG, LG)] * drow[b, e, pl.ds(a * LG, LG)]
                    for a in range(4)]
            for j in range(4, NSEG):
                a = j & 3
                accs[a] = accs[a] + (srow[b, e, pl.ds(j * LG, LG)]
                                     * drow[b, e, pl.ds(j * LG, LG)])
            acc = (accs[0] + accs[1]) + (accs[2] + accs[3])
            for p in perms:
                acc = acc + _permute(acc, p)
            gvec = jnp.where(lane == e, acc, gvec)
        outv[pl.ds(g * C, LG)] = 1.0 / (1.0 + jnp.exp(-gvec))

    start(0, 0)
    start(1, 1)

    def outer(t, carry):
        for b in range(2):
            g = 2 * t + b
            wait(g, b)
            compute(g, b)

            @pl.when(g + 2 < NCHUNK)
            def _():
                start(g + 2, b)
        return carry

    lax.fori_loop(0, NCHUNK // 2, outer, 0)
    pltpu.sync_copy(outv, out_hbm.at[pl.ds(base, EPW)])


@jax.jit
def _decode(z, src, dst):
    mesh = plsc.VectorSubcoreMesh(core_axis_name="c", subcore_axis_name="s")
    f = pl.kernel(
        _body,
        mesh=mesh,
        out_type=jax.ShapeDtypeStruct((PAD_E,), jnp.float32),
        scratch_types=[
            pltpu.VMEM((EPW,), jnp.int32),
            pltpu.VMEM((EPW,), jnp.int32),
            pltpu.VMEM((2, C, D), jnp.float32),
            pltpu.VMEM((2, C, D), jnp.float32),
            pltpu.VMEM((EPW,), jnp.float32),
            pltpu.SemaphoreType.DMA,
            pltpu.SemaphoreType.DMA,
            pltpu.SemaphoreType.DMA,
            pltpu.SemaphoreType.DMA,
        ],
    )
    return f(z, src, dst)


def kernel(z, edge_index):
    src = jnp.pad(edge_index[0].astype(jnp.int32), (0, PAD_E - N_EDGES))
    dst = jnp.pad(edge_index[1].astype(jnp.int32), (0, PAD_E - N_EDGES))
    return _decode(z, src, dst)[:N_EDGES]


# speedup vs baseline: 2.4271x; 1.8895x over previous
import jax
import jax.numpy as jnp
from jax import lax
from jax.experimental import pallas as pl
from jax.experimental.pallas import tpu as pltpu
from jax.experimental.pallas import tpu_sc as plsc

N_NODES = 10000
D = 256
DH = D // 2
N_EDGES = 160000
NC = 2
NS = 16
PAD_N = 10240
NPT = PAD_N // NS
EPT = 10240
PAD_E = EPT * NS
C = 16
NCHUNK = EPT // C
LG = 16
NSEG = DH // LG


def _permute(x, idx):
    dnums = lax.GatherDimensionNumbers(
        offset_dims=(), collapsed_slice_dims=(0,), start_index_map=(0,))
    return lax.gather(x, idx[:, None], dnums, (1,),
                      mode=lax.GatherScatterMode.PROMISE_IN_BOUNDS)


NBUF = 2


def _body(zs_hbm, src_hbm, dst_hbm, out_hbm,
          zsh, sidx, didx, srow, drow, outv,
          sem_s0, sem_s1, sem_s2, sem_s3, sem_d0, sem_d1, sem_d2, sem_d3):
    cid = lax.axis_index("c")
    sid = lax.axis_index("s")
    base = sid * EPT

    pltpu.sync_copy(zs_hbm.at[cid, pl.ds(sid * NPT, NPT)],
                    zsh.at[pl.ds(sid * NPT, NPT)])
    pltpu.sync_copy(src_hbm.at[pl.ds(base, EPT)], sidx)
    pltpu.sync_copy(dst_hbm.at[pl.ds(base, EPT)], didx)
    plsc.subcore_barrier()

    lane = lax.iota(jnp.int32, LG)
    xors = [lane ^ s for s in (8, 4, 2, 1)]
    masks = [(lane & s) == 0 for s in (8, 4, 2, 1)]
    sems = ((sem_s0, sem_d0), (sem_s1, sem_d1),
            (sem_s2, sem_d2), (sem_s3, sem_d3))

    def start(g, b):
        pltpu.async_copy(zsh.at[sidx.at[pl.ds(g * C, C)]], srow.at[b],
                         sems[b][0])
        pltpu.async_copy(zsh.at[didx.at[pl.ds(g * C, C)]], drow.at[b],
                         sems[b][1])

    def wait(g, b):
        pltpu.make_async_copy(zsh.at[sidx.at[pl.ds(g * C, C)]], srow.at[b],
                              sems[b][0]).wait()
        pltpu.make_async_copy(zsh.at[didx.at[pl.ds(g * C, C)]], drow.at[b],
                              sems[b][1]).wait()

    def compute(g, b):
        def edge_acc(e):
            accs = [None] * 2
            for j in range(NSEG):
                prod = (srow[b, e, pl.ds(j * LG, LG)]
                        * drow[b, e, pl.ds(j * LG, LG)])
                a = j & 1
                accs[a] = prod if accs[a] is None else accs[a] + prod
            return accs[0] + accs[1]

        def merge(a, u, lvl):
            ta = a + _permute(a, xors[lvl])
            tb = u + _permute(u, xors[lvl])
            return jnp.where(masks[lvl], ta, tb)

        v1 = [merge(edge_acc(i), edge_acc(i + 8), 0) for i in range(8)]
        v2 = [merge(v1[i], v1[i + 4], 1) for i in range(4)]
        v3 = [merge(v2[i], v2[i + 2], 2) for i in range(2)]
        outv[pl.ds(g * C, LG)] = merge(v3[0], v3[1], 3)

    for b in range(NBUF):
        start(b, b)

    def outer(t, carry):
        for b in range(NBUF):
            g = NBUF * t + b
            wait(g, b)
            compute(g, b)

            @pl.when(g + NBUF < NCHUNK)
            def _():
                start(g + NBUF, b)
        return carry

    lax.fori_loop(0, NCHUNK // NBUF, outer, 0)
    pltpu.sync_copy(outv, out_hbm.at[cid, pl.ds(base, EPT)])


@jax.jit
def _partial_dots(zsplit, src, dst):
    mesh = plsc.VectorSubcoreMesh(core_axis_name="c", subcore_axis_name="s")
    f = pl.kernel(
        _body,
        mesh=mesh,
        out_type=jax.ShapeDtypeStruct((NC, PAD_E), jnp.float32),
        scratch_types=[
            pltpu.VMEM_SHARED((PAD_N, DH), jnp.float32),
            pltpu.VMEM((EPT,), jnp.int32),
            pltpu.VMEM((EPT,), jnp.int32),
            pltpu.VMEM((NBUF, C, DH), jnp.float32),
            pltpu.VMEM((NBUF, C, DH), jnp.float32),
            pltpu.VMEM((EPT,), jnp.float32),
            pltpu.SemaphoreType.DMA,
            pltpu.SemaphoreType.DMA,
            pltpu.SemaphoreType.DMA,
            pltpu.SemaphoreType.DMA,
            pltpu.SemaphoreType.DMA,
            pltpu.SemaphoreType.DMA,
            pltpu.SemaphoreType.DMA,
            pltpu.SemaphoreType.DMA,
        ],
    )
    return f(zsplit, src, dst)


def _combine_body(p0_ref, p1_ref, o_ref):
    v = p0_ref[...] + p1_ref[...]
    o_ref[...] = 1.0 / (1.0 + jnp.exp(-v))


@jax.jit
def _combine(p0, p1):
    rows = PAD_E // 128
    f = pl.pallas_call(
        _combine_body,
        out_shape=jax.ShapeDtypeStruct((rows, 128), jnp.float32),
        grid=(rows // 256,),
        in_specs=[pl.BlockSpec((256, 128), lambda i: (i, 0)),
                  pl.BlockSpec((256, 128), lambda i: (i, 0))],
        out_specs=pl.BlockSpec((256, 128), lambda i: (i, 0)),
    )
    return f(p0.reshape(rows, 128), p1.reshape(rows, 128))


def kernel(z, edge_index):
    zpad = jnp.pad(z, ((0, PAD_N - N_NODES), (0, 0)))
    zsplit = jnp.stack([zpad[:, :DH], zpad[:, DH:]])
    src = jnp.pad(edge_index[0].astype(jnp.int32), (0, PAD_E - N_EDGES))
    dst = jnp.pad(edge_index[1].astype(jnp.int32), (0, PAD_E - N_EDGES))
    partial = _partial_dots(zsplit, src, dst)
    out = _combine(partial[0], partial[1])
    return out.reshape(PAD_E)[:N_EDGES]


# speedup vs baseline: 2.7967x; 1.1523x over previous
import jax
import jax.numpy as jnp
from jax import lax
from jax.experimental import pallas as pl
from jax.experimental.pallas import tpu as pltpu
from jax.experimental.pallas import tpu_sc as plsc

N_NODES = 10000
D = 256
DH = D // 2
N_EDGES = 160000
NC = 2
NS = 16
PAD_N = 10240
NPT = PAD_N // NS
EPT = 10240
PAD_E = EPT * NS
C = 16
NCHUNK = EPT // C
LG = 16
NSEG = DH // LG


def _permute(x, idx):
    dnums = lax.GatherDimensionNumbers(
        offset_dims=(), collapsed_slice_dims=(0,), start_index_map=(0,))
    return lax.gather(x, idx[:, None], dnums, (1,),
                      mode=lax.GatherScatterMode.PROMISE_IN_BOUNDS)


NBUF = 2


def _body(zs_hbm, src_hbm, dst_hbm, out_hbm,
          zsh, sidx, didx, srow, drow, outv,
          sem_s0, sem_s1, sem_d0, sem_d1):
    cid = lax.axis_index("c")
    sid = lax.axis_index("s")
    base = sid * EPT

    pltpu.sync_copy(zs_hbm.at[cid, pl.ds(sid * NPT, NPT)],
                    zsh.at[pl.ds(sid * NPT, NPT)])
    pltpu.sync_copy(src_hbm.at[pl.ds(base, EPT)], sidx)
    pltpu.sync_copy(dst_hbm.at[pl.ds(base, EPT)], didx)
    plsc.subcore_barrier()

    lane = lax.iota(jnp.int32, LG)
    perms = [(lane + s) & (LG - 1) for s in (8, 4, 2, 1)]
    sems = ((sem_s0, sem_d0), (sem_s1, sem_d1))

    def start(g, b):
        pltpu.async_copy(zsh.at[sidx.at[pl.ds(g * C, C)]], srow.at[b],
                         sems[b][0])
        pltpu.async_copy(zsh.at[didx.at[pl.ds(g * C, C)]], drow.at[b],
                         sems[b][1])

    def wait(g, b):
        pltpu.make_async_copy(zsh.at[sidx.at[pl.ds(g * C, C)]], srow.at[b],
                              sems[b][0]).wait()
        pltpu.make_async_copy(zsh.at[didx.at[pl.ds(g * C, C)]], drow.at[b],
                              sems[b][1]).wait()

    def compute(g, b):
        gvec = jnp.zeros((LG,), jnp.float32)
        for e in range(C):
            accs = [None] * 2
            for j in range(NSEG):
                prod = (srow[b, e, pl.ds(j * LG, LG)]
                        * drow[b, e, pl.ds(j * LG, LG)])
                a = j & 1
                accs[a] = prod if accs[a] is None else accs[a] + prod
            acc = accs[0] + accs[1]
            for p in perms:
                acc = acc + _permute(acc, p)
            gvec = jnp.where(lane == e, acc, gvec)
        outv[pl.ds(g * C, LG)] = gvec

    for b in range(NBUF):
        start(b, b)

    def outer(t, carry):
        for b in range(NBUF):
            g = NBUF * t + b
            wait(g, b)
            compute(g, b)

            @pl.when(g + NBUF < NCHUNK)
            def _():
                start(g + NBUF, b)
        return carry

    lax.fori_loop(0, NCHUNK // NBUF, outer, 0)
    pltpu.sync_copy(outv, out_hbm.at[cid, pl.ds(base, EPT)])


@jax.jit
def _partial_dots(zsplit, src, dst):
    mesh = plsc.VectorSubcoreMesh(core_axis_name="c", subcore_axis_name="s")
    f = pl.kernel(
        _body,
        mesh=mesh,
        out_type=jax.ShapeDtypeStruct((NC, PAD_E), jnp.float32),
        scratch_types=[
            pltpu.VMEM_SHARED((PAD_N, DH), jnp.float32),
            pltpu.VMEM((EPT,), jnp.int32),
            pltpu.VMEM((EPT,), jnp.int32),
            pltpu.VMEM((NBUF, C, DH), jnp.float32),
            pltpu.VMEM((NBUF, C, DH), jnp.float32),
            pltpu.VMEM((EPT,), jnp.float32),
            pltpu.SemaphoreType.DMA,
            pltpu.SemaphoreType.DMA,
            pltpu.SemaphoreType.DMA,
            pltpu.SemaphoreType.DMA,
        ],
    )
    return f(zsplit, src, dst)


def _combine_body(p0_ref, p1_ref, o_ref):
    v = p0_ref[...] + p1_ref[...]
    o_ref[...] = 1.0 / (1.0 + jnp.exp(-v))


@jax.jit
def _combine(p0, p1):
    rows = PAD_E // 128
    f = pl.pallas_call(
        _combine_body,
        out_shape=jax.ShapeDtypeStruct((rows, 128), jnp.float32),
        grid=(rows // 256,),
        in_specs=[pl.BlockSpec((256, 128), lambda i: (i, 0)),
                  pl.BlockSpec((256, 128), lambda i: (i, 0))],
        out_specs=pl.BlockSpec((256, 128), lambda i: (i, 0)),
    )
    return f(p0.reshape(rows, 128), p1.reshape(rows, 128))


def kernel(z, edge_index):
    zpad = jnp.pad(z, ((0, PAD_N - N_NODES), (0, 0)))
    zsplit = jnp.stack([zpad[:, :DH], zpad[:, DH:]])
    src = jnp.pad(edge_index[0].astype(jnp.int32), (0, PAD_E - N_EDGES))
    dst = jnp.pad(edge_index[1].astype(jnp.int32), (0, PAD_E - N_EDGES))
    partial = _partial_dots(zsplit, src, dst)
    out = _combine(partial[0], partial[1])
    return out.reshape(PAD_E)[:N_EDGES]
